# Initial kernel scaffold; baseline (speedup 1.0000x reference)
import jax
import jax.numpy as jnp
from jax.experimental import pallas as pl


def kernel(Z, table):
    raise NotImplementedError("write your pallas kernel here")


# speedup vs baseline: 4.7850x; 4.7850x over previous
import functools

import jax
import jax.numpy as jnp
from jax import lax
from jax.experimental import pallas as pl
from jax.experimental.pallas import tpu as pltpu
from jax.experimental.pallas import tpu_sc as plsc

MAX_Z = 119
DIM = 64
N_ATOMS = 1_000_000
NC = 2
NS = 16
NW = NC * NS
CHUNK = 112
CHUNKS = 279
PER_W = CHUNK * CHUNKS
TAIL = N_ATOMS - NW * PER_W
NBUF = 3


@functools.partial(
    pl.kernel,
    out_type=jax.ShapeDtypeStruct((N_ATOMS, DIM), jnp.float32),
    mesh=plsc.VectorSubcoreMesh(core_axis_name="c", subcore_axis_name="s"),
    scratch_types=[
        pltpu.VMEM((PER_W,), jnp.int32),
        pltpu.VMEM((TAIL,), jnp.int32),
        pltpu.VMEM((NBUF, CHUNK, DIM), jnp.float32),
        pltpu.VMEM((TAIL, DIM), jnp.float32),
        pltpu.VMEM((MAX_Z, DIM), jnp.float32),
        pltpu.VMEM_SHARED((MAX_Z, DIM), jnp.float32),
        pltpu.SemaphoreType.DMA,
        pltpu.SemaphoreType.DMA,
        pltpu.SemaphoreType.DMA,
    ],
)
def _embed(z_hbm, table_hbm, out_hbm, idx_v, tidx_v, rows_v, trow_v,
           table_v, table_s, gsem0, gsem1, gsem2):
    cid = lax.axis_index("c")
    sid = lax.axis_index("s")
    wid = sid * NC + cid

    @pl.when(sid == 0)
    def _():
        pltpu.sync_copy(table_hbm, table_v)
        pltpu.sync_copy(table_v, table_s)

    plsc.subcore_barrier()

    base = pl.multiple_of(wid * PER_W, 8)

    @pl.when(wid == NW - 1)
    def _():
        pltpu.sync_copy(z_hbm.at[pl.ds(NW * PER_W, TAIL)], tidx_v)
        pltpu.async_copy(table_s.at[tidx_v], trow_v, gsem0).wait()
        pltpu.sync_copy(trow_v, out_hbm.at[pl.ds(NW * PER_W, TAIL)])

    pltpu.sync_copy(z_hbm.at[pl.ds(base, PER_W)], idx_v)

    gsems = (gsem0, gsem1, gsem2)

    def start_gather(j, slot):
        pltpu.async_copy(
            table_s.at[idx_v.at[pl.ds(j * CHUNK, CHUNK)]],
            rows_v.at[slot], gsems[slot])

    def wait_gather(slot):
        pltpu.make_async_copy(
            table_s.at[idx_v.at[pl.ds(0, CHUNK)]],
            rows_v.at[slot], gsems[slot]).wait()

    start_gather(0, 0)
    start_gather(1, 1)

    def body(g, _):
        for b in range(NBUF):
            j = g * NBUF + b

            if b == 0:
                start_gather(j + 2, (b + 2) % NBUF)
            else:
                @pl.when(g + 1 < CHUNKS // NBUF)
                def _():
                    start_gather(j + 2, (b + 2) % NBUF)

            wait_gather(b)
            pltpu.sync_copy(
                rows_v.at[b],
                out_hbm.at[pl.ds(pl.multiple_of(base + j * CHUNK, 8), CHUNK)])
        return 0

    lax.fori_loop(0, CHUNKS // NBUF, body, 0)


def kernel(Z, table):
    return _embed(Z.astype(jnp.int32), table)


# speedup vs baseline: 4.8408x; 1.0117x over previous
import functools

import jax
import jax.numpy as jnp
from jax import lax
from jax.experimental import pallas as pl
from jax.experimental.pallas import tpu as pltpu
from jax.experimental.pallas import tpu_sc as plsc

MAX_Z = 119
DIM = 64
N_ATOMS = 1_000_000
NC = 2
NS = 16
NW = NC * NS
CHUNK = 168
CHUNKS = 186
PER_W = CHUNK * CHUNKS
TAIL = N_ATOMS - NW * PER_W
NBUF = 3


@functools.partial(
    pl.kernel,
    out_type=jax.ShapeDtypeStruct((N_ATOMS, DIM), jnp.float32),
    mesh=plsc.VectorSubcoreMesh(core_axis_name="c", subcore_axis_name="s"),
    scratch_types=[
        pltpu.VMEM((PER_W,), jnp.int32),
        pltpu.VMEM((TAIL,), jnp.int32),
        pltpu.VMEM((NBUF, CHUNK, DIM), jnp.float32),
        pltpu.VMEM((TAIL, DIM), jnp.float32),
        pltpu.VMEM((MAX_Z, DIM), jnp.float32),
        pltpu.VMEM_SHARED((MAX_Z, DIM), jnp.float32),
        pltpu.SemaphoreType.DMA,
        pltpu.SemaphoreType.DMA,
        pltpu.SemaphoreType.DMA,
        pltpu.SemaphoreType.DMA,
        pltpu.SemaphoreType.DMA,
        pltpu.SemaphoreType.DMA,
    ],
)
def _embed(z_hbm, table_hbm, out_hbm, idx_v, tidx_v, rows_v, trow_v,
           table_v, table_s, gsem0, gsem1, gsem2, wsem0, wsem1, wsem2):
    cid = lax.axis_index("c")
    sid = lax.axis_index("s")
    wid = sid * NC + cid

    @pl.when(sid == 0)
    def _():
        pltpu.sync_copy(table_hbm, table_v)
        pltpu.sync_copy(table_v, table_s)

    plsc.subcore_barrier()

    base = pl.multiple_of(wid * PER_W, 8)

    @pl.when(wid == NW - 1)
    def _():
        pltpu.sync_copy(z_hbm.at[pl.ds(NW * PER_W, TAIL)], tidx_v)
        pltpu.async_copy(table_s.at[tidx_v], trow_v, gsem0).wait()
        pltpu.sync_copy(trow_v, out_hbm.at[pl.ds(NW * PER_W, TAIL)])

    pltpu.sync_copy(z_hbm.at[pl.ds(base, PER_W)], idx_v)

    gsems = (gsem0, gsem1, gsem2)
    wsems = (wsem0, wsem1, wsem2)

    def out_slice(j):
        return out_hbm.at[pl.ds(pl.multiple_of(base + j * CHUNK, 8), CHUNK)]

    def start_gather(j, slot):
        pltpu.async_copy(
            table_s.at[idx_v.at[pl.ds(j * CHUNK, CHUNK)]],
            rows_v.at[slot], gsems[slot])

    def wait_gather(j, slot):
        pltpu.make_async_copy(
            table_s.at[idx_v.at[pl.ds(j * CHUNK, CHUNK)]],
            rows_v.at[slot], gsems[slot]).wait()

    def start_write(j, slot):
        pltpu.async_copy(rows_v.at[slot], out_slice(j), wsems[slot])

    def wait_write(j, slot):
        pltpu.make_async_copy(
            rows_v.at[slot], out_slice(j), wsems[slot]).wait()

    start_gather(0, 0)
    start_gather(1, 1)

    def body(g, _):
        for b in range(NBUF):
            j = g * NBUF + b
            prev = (b + NBUF - 1) % NBUF

            wait_gather(j, b)
            start_write(j, b)
            wait_write(j, b)

            if b == 0:
                start_gather(j + 2, prev)
            else:
                @pl.when(g + 1 < CHUNKS // NBUF)
                def _():
                    start_gather(j + 2, prev)
        return 0

    lax.fori_loop(0, CHUNKS // NBUF, body, 0)


def kernel(Z, table):
    return _embed(Z.astype(jnp.int32), table)
